# baseline (speedup 1.0000x reference)
import functools

import jax
import jax.numpy as jnp
from jax import lax
from jax.experimental import pallas as pl
from jax.experimental.pallas import tpu as pltpu
from jax.experimental.pallas import tpu_sc as plsc

D = 64
CHUNK = 128
NC = 2
NS = 16
NW = NC * NS
NCHUNKS = 25
BPW = CHUNK * NCHUNKS
B_PAD = NW * BPW


@functools.cache
def _build():
    mesh = plsc.VectorSubcoreMesh(core_axis_name="c", subcore_axis_name="s")

    @functools.partial(
        pl.kernel,
        mesh=mesh,
        out_type=jax.ShapeDtypeStruct((B_PAD, D), jnp.float32),
        scratch_types=[
            pltpu.VMEM((NCHUNKS, CHUNK), jnp.int32),
            pltpu.VMEM((CHUNK, D), jnp.float32),
            pltpu.SemaphoreType.DMA,
        ],
        compiler_params=pltpu.CompilerParams(use_tc_tiling_on_sc=False),
    )
    def gather_kernel(table_hbm, idx_hbm, out_hbm, idx_v, rows_v, sem):
        wid = lax.axis_index("s") * NC + lax.axis_index("c")
        base = wid * BPW
        pltpu.sync_copy(idx_hbm.at[wid], idx_v)

        def body(j, carry):
            pltpu.async_copy(table_hbm.at[idx_v.at[j]], rows_v, sem).wait()
            pltpu.sync_copy(rows_v, out_hbm.at[pl.ds(base + j * CHUNK, CHUNK)])
            return carry

        lax.fori_loop(0, NCHUNKS, body, 0)

    return gather_kernel


def kernel(x, emb_weight):
    idx = x[:, 0].astype(jnp.int32)
    n = idx.shape[0]
    idx_p = jnp.pad(idx, (0, B_PAD - n))
    idx_r = idx_p.reshape(NW, NCHUNKS, CHUNK)
    out = _build()(emb_weight, idx_r)
    return out[:n]


# speedup vs baseline: 1.7091x; 1.7091x over previous
import functools

import jax
import jax.numpy as jnp
from jax import lax
from jax.experimental import pallas as pl
from jax.experimental.pallas import tpu as pltpu
from jax.experimental.pallas import tpu_sc as plsc

D = 64
CHUNK = 125
NC = 2
NS = 16
NW = NC * NS
NCHUNKS = 25
DEPTH = 5
BPW = CHUNK * NCHUNKS
B = NW * BPW


@functools.cache
def _build():
    mesh = plsc.VectorSubcoreMesh(core_axis_name="c", subcore_axis_name="s")

    row_bufs = [pltpu.VMEM((CHUNK, D), jnp.float32) for _ in range(DEPTH)]

    @functools.partial(
        pl.kernel,
        mesh=mesh,
        out_type=jax.ShapeDtypeStruct((B, D), jnp.float32),
        scratch_types=[pltpu.VMEM((NCHUNKS, CHUNK), jnp.int32)]
        + row_bufs
        + [pltpu.SemaphoreType.DMA] * DEPTH
        + [pltpu.SemaphoreType.DMA] * DEPTH,
        compiler_params=pltpu.CompilerParams(use_tc_tiling_on_sc=False),
    )
    def gather_kernel(table_hbm, idx_hbm, out_hbm, idx_v, *scratch):
        rows = scratch[:DEPTH]
        gsem = scratch[DEPTH : 2 * DEPTH]
        wsem = scratch[2 * DEPTH : 3 * DEPTH]
        wid = lax.axis_index("s") * NC + lax.axis_index("c")
        base = wid * BPW
        pltpu.sync_copy(idx_hbm.at[wid], idx_v)
        for b in range(DEPTH):
            pltpu.async_copy(table_hbm.at[idx_v.at[b]], rows[b], gsem[b])

        def body(i, carry):
            for b in range(DEPTH):
                j = DEPTH * i + b
                pltpu.make_async_copy(
                    table_hbm.at[idx_v.at[0]], rows[b], gsem[b]
                ).wait()
                pltpu.async_copy(
                    rows[b], out_hbm.at[pl.ds(base + j * CHUNK, CHUNK)], wsem[b]
                )

                @pl.when(i < NCHUNKS // DEPTH - 1)
                def _():
                    pltpu.make_async_copy(
                        rows[b], out_hbm.at[pl.ds(base, CHUNK)], wsem[b]
                    ).wait()
                    pltpu.async_copy(
                        table_hbm.at[idx_v.at[j + DEPTH]], rows[b], gsem[b]
                    )

            return carry

        lax.fori_loop(0, NCHUNKS // DEPTH, body, 0)
        for b in range(DEPTH):
            pltpu.make_async_copy(
                rows[b], out_hbm.at[pl.ds(base, CHUNK)], wsem[b]
            ).wait()

    return gather_kernel


def kernel(x, emb_weight):
    idx = x[:, 0].astype(jnp.int32)
    idx_r = idx.reshape(NW, NCHUNKS, CHUNK)
    return _build()(emb_weight, idx_r)
